# baseline (speedup 1.0000x reference)
import functools

import jax
import jax.numpy as jnp
from jax import lax
from jax.experimental import pallas as pl
from jax.experimental.pallas import tpu as pltpu
from jax.experimental.pallas import tpu_sc as plsc

_IGNORE = 255
_N, _C = 65536, 512
_L = 16
_NC, _NS = 2, 16
_NW = _NC * _NS
_BPW = _N // _NW
_NCHUNK = 4
_CH = _BPW // _NCHUNK
_CHG = _CH // _L

_mesh = plsc.VectorSubcoreMesh(
    core_axis_name="c", subcore_axis_name="s", num_cores=_NC, num_subcores=_NS
)


@functools.partial(
    pl.kernel,
    out_type=jax.ShapeDtypeStruct((_NW, 2 * _L), jnp.float32),
    mesh=_mesh,
    scratch_types=[
        pltpu.VMEM((_BPW,), jnp.int32),
        pltpu.VMEM((_BPW,), jnp.float32),
        pltpu.VMEM((2 * _L,), jnp.float32),
        [pltpu.SemaphoreType.DMA] * _NCHUNK,
    ],
)
def _sc_partials(logits_hbm, tgt_hbm, part_hbm,
                 tgt_v, val_v, acc_v, sems):
    wid = lax.axis_index("s") * _NC + lax.axis_index("c")
    base = wid * _BPW
    pltpu.sync_copy(tgt_hbm.at[pl.ds(base, _BPW)], tgt_v)

    lane = lax.iota(jnp.int32, 16)
    lane_off = ((lane >> 3) << 12) + ((lane & 7) << 7)

    def gather_body(k, j, _):
        jj = j * _L
        t = tgt_v[pl.ds(jj, _L)]
        phys = ((base + jj) << 9) + (lane_off + (((t >> 7) << 10) + (t & 127)))
        pltpu.async_copy(logits_hbm.at[phys], val_v.at[pl.ds(jj, _L)], sems[k])
        return 0

    for k in range(_NCHUNK):
        lax.fori_loop(k * _CHG, (k + 1) * _CHG,
                      functools.partial(gather_body, k), 0, unroll=4)

    copies = [
        pltpu.make_async_copy(
            logits_hbm.at[pl.ds(0, _CH)],
            val_v.at[pl.ds(k * _CH, _CH)],
            sems[k],
        )
        for k in range(_NCHUNK)
    ]

    one = jnp.ones((_L,), jnp.float32)
    zero = jnp.zeros((_L,), jnp.float32)

    def acc_body(j, carry):
        s, c = carry
        jj = j * _L
        t = tgt_v[pl.ds(jj, _L)]
        g = val_v[pl.ds(jj, _L)]
        m = t != _IGNORE
        d = jnp.where(m, 1.0 - g, 0.0)
        return s + d * d, c + jnp.where(m, one, zero)

    s = zero
    c = zero
    for k in range(_NCHUNK):
        copies[k].wait()
        s, c = lax.fori_loop(k * _CHG, (k + 1) * _CHG, acc_body, (s, c),
                             unroll=4)

    acc_v[pl.ds(0, _L)] = s
    acc_v[pl.ds(_L, _L)] = c
    pltpu.sync_copy(acc_v, part_hbm.at[wid])


def _finalize_body(part_ref, out_ref):
    p = part_ref[...]
    s = jnp.sum(p[:, :_L])
    c = jnp.sum(p[:, _L:])
    out_ref[...] = (s / jnp.maximum(c, 1.0)).reshape(1, 1)


def kernel(contrast_logits, contrast_target):
    logits_flat = (
        contrast_logits.reshape(_N // 8, 8, _C // 128, 128)
        .transpose(0, 2, 1, 3)
        .reshape(-1)
    )
    parts = _sc_partials(logits_flat, contrast_target)
    out = pl.pallas_call(
        _finalize_body,
        out_shape=jax.ShapeDtypeStruct((1, 1), jnp.float32),
    )(parts)
    return out[0, 0]


# speedup vs baseline: 1.0102x; 1.0102x over previous
import functools

import jax
import jax.numpy as jnp
from jax import lax
from jax.experimental import pallas as pl
from jax.experimental.pallas import tpu as pltpu
from jax.experimental.pallas import tpu_sc as plsc

_IGNORE = 255
_N, _C = 65536, 512
_L = 16
_NC, _NS = 2, 16
_NW = _NC * _NS
_BPW = _N // _NW
_NCHUNK = 4
_CH = _BPW // _NCHUNK
_CHG = _CH // _L

_mesh = plsc.VectorSubcoreMesh(
    core_axis_name="c", subcore_axis_name="s", num_cores=_NC, num_subcores=_NS
)


@functools.partial(
    pl.kernel,
    out_type=jax.ShapeDtypeStruct((_NW, 2 * _L), jnp.float32),
    mesh=_mesh,
    scratch_types=[
        pltpu.VMEM((_BPW,), jnp.int32),
        pltpu.VMEM((_BPW,), jnp.float32),
        pltpu.VMEM((2 * _L,), jnp.float32),
        [pltpu.SemaphoreType.DMA] * _NCHUNK,
    ],
)
def _sc_partials(logits_hbm, tgt_hbm, part_hbm,
                 tgt_v, val_v, acc_v, sems):
    wid = lax.axis_index("s") * _NC + lax.axis_index("c")
    base = wid * _BPW
    pltpu.sync_copy(tgt_hbm.at[pl.ds(base, _BPW)], tgt_v)

    lane = lax.iota(jnp.int32, 16)
    lane_off = ((lane >> 3) << 12) + ((lane & 7) << 7)

    def gather_body(k, j, _):
        jj = j * _L
        t = tgt_v[pl.ds(jj, _L)]
        phys = ((base + jj) << 9) + (lane_off + (((t >> 7) << 10) + (t & 127)))
        pltpu.async_copy(logits_hbm.at[phys], val_v.at[pl.ds(jj, _L)], sems[k])
        return 0

    for k in range(_NCHUNK):
        lax.fori_loop(k * _CHG, (k + 1) * _CHG,
                      functools.partial(gather_body, k), 0, unroll=2)

    copies = [
        pltpu.make_async_copy(
            logits_hbm.at[pl.ds(0, _CH)],
            val_v.at[pl.ds(k * _CH, _CH)],
            sems[k],
        )
        for k in range(_NCHUNK)
    ]

    one = jnp.ones((_L,), jnp.float32)
    zero = jnp.zeros((_L,), jnp.float32)

    def acc_body(j, carry):
        s, c = carry
        jj = j * _L
        t = tgt_v[pl.ds(jj, _L)]
        g = val_v[pl.ds(jj, _L)]
        m = t != _IGNORE
        d = jnp.where(m, 1.0 - g, 0.0)
        return s + d * d, c + jnp.where(m, one, zero)

    s = zero
    c = zero
    for k in range(_NCHUNK):
        copies[k].wait()
        s, c = lax.fori_loop(k * _CHG, (k + 1) * _CHG, acc_body, (s, c),
                             unroll=2)

    acc_v[pl.ds(0, _L)] = s
    acc_v[pl.ds(_L, _L)] = c
    pltpu.sync_copy(acc_v, part_hbm.at[wid])


def _finalize_body(part_ref, out_ref):
    p = part_ref[...]
    s = jnp.sum(p[:, :_L])
    c = jnp.sum(p[:, _L:])
    out_ref[...] = (s / jnp.maximum(c, 1.0)).reshape(1, 1)


def kernel(contrast_logits, contrast_target):
    logits_flat = (
        contrast_logits.reshape(_N // 8, 8, _C // 128, 128)
        .transpose(0, 2, 1, 3)
        .reshape(-1)
    )
    parts = _sc_partials(logits_flat, contrast_target)
    out = pl.pallas_call(
        _finalize_body,
        out_shape=jax.ShapeDtypeStruct((1, 1), jnp.float32),
    )(parts)
    return out[0, 0]


# speedup vs baseline: 1.0117x; 1.0015x over previous
import functools

import jax
import jax.numpy as jnp
from jax import lax
from jax.experimental import pallas as pl
from jax.experimental.pallas import tpu as pltpu
from jax.experimental.pallas import tpu_sc as plsc

_IGNORE = 255
_N, _C = 65536, 512
_L = 16
_NC, _NS = 2, 16
_NW = _NC * _NS
_BPW = _N // _NW
_NCHUNK = 4
_CH = _BPW // _NCHUNK
_CHG = _CH // _L

_mesh = plsc.VectorSubcoreMesh(
    core_axis_name="c", subcore_axis_name="s", num_cores=_NC, num_subcores=_NS
)


@functools.partial(
    pl.kernel,
    out_type=jax.ShapeDtypeStruct((_NW, 2 * _L), jnp.float32),
    mesh=_mesh,
    scratch_types=[
        pltpu.VMEM((_BPW,), jnp.int32),
        pltpu.VMEM((_BPW,), jnp.float32),
        pltpu.VMEM((2 * _L,), jnp.float32),
        [pltpu.SemaphoreType.DMA] * _NCHUNK,
        [pltpu.SemaphoreType.DMA] * _NCHUNK,
    ],
)
def _sc_partials(logits_hbm, tgt_hbm, part_hbm,
                 tgt_v, val_v, acc_v, sems, tsems):
    wid = lax.axis_index("s") * _NC + lax.axis_index("c")
    base = wid * _BPW
    tgt_copies = [
        pltpu.async_copy(
            tgt_hbm.at[pl.ds(base + k * _CH, _CH)],
            tgt_v.at[pl.ds(k * _CH, _CH)],
            tsems[k],
        )
        for k in range(_NCHUNK)
    ]

    lane = lax.iota(jnp.int32, 16)
    lane_off = ((lane >> 3) << 12) + ((lane & 7) << 7)

    def gather_body(k, j, _):
        jj = j * _L
        t = tgt_v[pl.ds(jj, _L)]
        phys = ((base + jj) << 9) + (lane_off + (((t >> 7) << 10) + (t & 127)))
        pltpu.async_copy(logits_hbm.at[phys], val_v.at[pl.ds(jj, _L)], sems[k])
        return 0

    for k in range(_NCHUNK):
        tgt_copies[k].wait()
        lax.fori_loop(k * _CHG, (k + 1) * _CHG,
                      functools.partial(gather_body, k), 0, unroll=2)

    copies = [
        pltpu.make_async_copy(
            logits_hbm.at[pl.ds(0, _CH)],
            val_v.at[pl.ds(k * _CH, _CH)],
            sems[k],
        )
        for k in range(_NCHUNK)
    ]

    one = jnp.ones((_L,), jnp.float32)
    zero = jnp.zeros((_L,), jnp.float32)

    def acc_body(j, carry):
        s, c = carry
        jj = j * _L
        t = tgt_v[pl.ds(jj, _L)]
        g = val_v[pl.ds(jj, _L)]
        m = t != _IGNORE
        d = jnp.where(m, 1.0 - g, 0.0)
        return s + d * d, c + jnp.where(m, one, zero)

    s = zero
    c = zero
    for k in range(_NCHUNK):
        copies[k].wait()
        s, c = lax.fori_loop(k * _CHG, (k + 1) * _CHG, acc_body, (s, c),
                             unroll=2)

    acc_v[pl.ds(0, _L)] = s
    acc_v[pl.ds(_L, _L)] = c
    pltpu.sync_copy(acc_v, part_hbm.at[wid])


def _finalize_body(part_ref, out_ref):
    p = part_ref[...]
    s = jnp.sum(p[:, :_L])
    c = jnp.sum(p[:, _L:])
    out_ref[...] = (s / jnp.maximum(c, 1.0)).reshape(1, 1)


def kernel(contrast_logits, contrast_target):
    logits_flat = (
        contrast_logits.reshape(_N // 8, 8, _C // 128, 128)
        .transpose(0, 2, 1, 3)
        .reshape(-1)
    )
    parts = _sc_partials(logits_flat, contrast_target)
    out = pl.pallas_call(
        _finalize_body,
        out_shape=jax.ShapeDtypeStruct((1, 1), jnp.float32),
    )(parts)
    return out[0, 0]
